# baseline (speedup 1.0000x reference)
import jax
import jax.numpy as jnp
from jax import lax
from jax.experimental import pallas as pl
from jax.experimental.pallas import tpu as pltpu

_B = 16384
_C = 1000
_BLK = 2048


def _bellman_t_block(savt_ref, act_ref, q_ref, outt_ref):
    i = pl.program_id(0)
    rows = lax.broadcasted_iota(jnp.int32, outt_ref.shape, 0)
    act_blk = act_ref[:, pl.ds(i * _BLK, _BLK)]
    q_blk = q_ref[:, pl.ds(i * _BLK, _BLK)]
    outt_ref[...] = jnp.where(rows == act_blk, q_blk, savt_ref[...])


def kernel(state_action_values, action, q_prime):
    savt = state_action_values.T
    act = action.astype(jnp.int32).reshape(1, _B)
    q2 = q_prime.reshape(1, _B)
    outt = pl.pallas_call(
        _bellman_t_block,
        grid=(_B // _BLK,),
        in_specs=[
            pl.BlockSpec((_C, _BLK), lambda i: (0, i)),
            pl.BlockSpec((1, _B), lambda i: (0, 0)),
            pl.BlockSpec((1, _B), lambda i: (0, 0)),
        ],
        out_specs=pl.BlockSpec((_C, _BLK), lambda i: (0, i)),
        out_shape=jax.ShapeDtypeStruct((_C, _B), jnp.float32),
        compiler_params=pltpu.CompilerParams(
            dimension_semantics=("arbitrary",),
        ),
    )(savt, act, q2)
    return outt.T


# speedup vs baseline: 1.0437x; 1.0437x over previous
import jax
import jax.numpy as jnp
from jax import lax
from jax.experimental import pallas as pl
from jax.experimental.pallas import tpu as pltpu

_B = 16384
_C = 1000
_BLK = 2048
_NCH = _B // _BLK
_NI = 3
_NO = 3


def _ring_body(savt_hbm, act_hbm, q_hbm, outt_hbm,
               act_v, q_v, ibufs, obufs, sem_small, in_sems, out_sems):
    def in_copy(g, b):
        return pltpu.make_async_copy(
            savt_hbm.at[:, pl.ds(g * _BLK, _BLK)], ibufs.at[b], in_sems.at[b])

    def out_copy(g, b):
        return pltpu.make_async_copy(
            obufs.at[b], outt_hbm.at[:, pl.ds(g * _BLK, _BLK)], out_sems.at[b])

    pltpu.make_async_copy(act_hbm, act_v, sem_small).start()
    pltpu.make_async_copy(q_hbm, q_v, sem_small).start()
    for b in range(_NI):
        in_copy(b, b).start()
    pltpu.make_async_copy(act_hbm, act_v, sem_small).wait()
    pltpu.make_async_copy(q_hbm, q_v, sem_small).wait()

    rows = lax.broadcasted_iota(jnp.int32, (_C, _BLK), 0)

    def step(g, carry):
        bi = lax.rem(g, _NI)
        bo = lax.rem(g, _NO)

        @pl.when(g >= _NO)
        def _():
            out_copy(g - _NO, bo).wait()

        in_copy(g, bi).wait()
        act_blk = act_v[:, pl.ds(g * _BLK, _BLK)]
        q_blk = q_v[:, pl.ds(g * _BLK, _BLK)]
        obufs[bo] = jnp.where(rows == act_blk, q_blk, ibufs[bi])
        out_copy(g, bo).start()

        @pl.when(g + _NI < _NCH)
        def _():
            in_copy(g + _NI, bi).start()

        return carry

    lax.fori_loop(0, _NCH, step, 0)

    for b in range(_NO):
        g = _NCH - _NO + b
        out_copy(g, lax.rem(jnp.int32(g), _NO)).wait()


def kernel(state_action_values, action, q_prime):
    savt = state_action_values.T
    act = action.astype(jnp.int32).reshape(1, _B)
    q2 = q_prime.reshape(1, _B)
    outt = pl.pallas_call(
        _ring_body,
        in_specs=[
            pl.BlockSpec(memory_space=pl.ANY),
            pl.BlockSpec(memory_space=pl.ANY),
            pl.BlockSpec(memory_space=pl.ANY),
        ],
        out_specs=pl.BlockSpec(memory_space=pl.ANY),
        out_shape=jax.ShapeDtypeStruct((_C, _B), jnp.float32),
        scratch_shapes=[
            pltpu.VMEM((1, _B), jnp.int32),
            pltpu.VMEM((1, _B), jnp.float32),
            pltpu.VMEM((_NI, _C, _BLK), jnp.float32),
            pltpu.VMEM((_NO, _C, _BLK), jnp.float32),
            pltpu.SemaphoreType.DMA,
            pltpu.SemaphoreType.DMA((_NI,)),
            pltpu.SemaphoreType.DMA((_NO,)),
        ],
    )(savt, act, q2)
    return outt.T
